# baseline (speedup 1.0000x reference)
import jax
import jax.numpy as jnp
from jax import lax
from jax.experimental import pallas as pl
from jax.experimental.pallas import tpu as pltpu
from jax.experimental.pallas import tpu_sc as plsc

B = 16384
EMBED_DIM = 32
NUM_BUCKETS = 1000
PAD_BUCKETS = 1024
NC, NS, L = 2, 16, 16
NW = NC * NS
CHUNK = B // NW
GATHER_W = 128
N_GATHERS = CHUNK // GATHER_W
TS_R = 2048


def _ts_body(ts_ref, blo_ref, bhi_ref, tab_ref, mean_ref, std_ref,
             emb_ref, norm_ref):
    ts = ts_ref[...]
    onehot = ((blo_ref[...] <= ts).astype(jnp.float32) -
              (bhi_ref[...] <= ts).astype(jnp.float32))
    emb_ref[...] = jnp.dot(onehot, tab_ref[...],
                           preferred_element_type=jnp.float32,
                           precision=lax.Precision.HIGHEST)
    norm_ref[...] = (ts - mean_ref[0, 0]) / std_ref[0, 0]


def _ts_embed(timestamp, ts_table, boundaries, ts_mean, ts_var):
    inf = jnp.full((PAD_BUCKETS - NUM_BUCKETS,), jnp.inf, jnp.float32)
    bhi = jnp.concatenate([boundaries, inf]).reshape(1, PAD_BUCKETS)
    blo = jnp.concatenate(
        [jnp.full((1,), -jnp.inf, jnp.float32), bhi[0, :PAD_BUCKETS - 1]]
    ).reshape(1, PAD_BUCKETS)
    tab = jnp.zeros((PAD_BUCKETS, EMBED_DIM), jnp.float32)
    tab = tab.at[:NUM_BUCKETS + 1].set(ts_table)
    mean = jnp.reshape(ts_mean, (1, 1)).astype(jnp.float32)
    std = jnp.reshape(jnp.sqrt(ts_var), (1, 1)).astype(jnp.float32)
    grid = B // TS_R
    return pl.pallas_call(
        _ts_body,
        grid=(grid,),
        in_specs=[
            pl.BlockSpec((TS_R, 1), lambda i: (i, 0)),
            pl.BlockSpec((1, PAD_BUCKETS), lambda i: (0, 0)),
            pl.BlockSpec((1, PAD_BUCKETS), lambda i: (0, 0)),
            pl.BlockSpec((PAD_BUCKETS, EMBED_DIM), lambda i: (0, 0)),
            pl.BlockSpec((1, 1), lambda i: (0, 0)),
            pl.BlockSpec((1, 1), lambda i: (0, 0)),
        ],
        out_specs=[
            pl.BlockSpec((TS_R, EMBED_DIM), lambda i: (i, 0)),
            pl.BlockSpec((TS_R, 1), lambda i: (i, 0)),
        ],
        out_shape=[
            jax.ShapeDtypeStruct((B, EMBED_DIM), jnp.float32),
            jax.ShapeDtypeStruct((B, 1), jnp.float32),
        ],
        compiler_params=pltpu.CompilerParams(
            dimension_semantics=("parallel",)),
    )(timestamp.reshape(B, 1).astype(jnp.float32), blo, bhi, tab, mean, std)


def _sc_body(uidx_hbm, utab_hbm, out_hbm, uidx_v, urows_v, gsem, osem):
    wid = lax.axis_index("s") * NC + lax.axis_index("c")
    base = wid * CHUNK

    pltpu.sync_copy(uidx_hbm.at[pl.ds(base, CHUNK)], uidx_v)
    copies = []
    for j in range(N_GATHERS):
        copies.append(pltpu.async_copy(
            utab_hbm.at[uidx_v.at[pl.ds(j * GATHER_W, GATHER_W)]],
            urows_v.at[pl.ds(j * GATHER_W, GATHER_W)], gsem))
    for c in copies:
        c.wait()
    out = pltpu.async_copy(urows_v, out_hbm.at[pl.ds(base, CHUNK)], osem)
    out.wait()


def kernel(user_idx, timestamp, user_table, ts_table, boundaries, ts_mean,
           ts_var):
    ts_emb, norm = _ts_embed(timestamp, ts_table, boundaries, ts_mean, ts_var)

    mesh = plsc.VectorSubcoreMesh(core_axis_name="c", subcore_axis_name="s")
    sc = pl.kernel(
        _sc_body,
        out_type=jax.ShapeDtypeStruct((B, EMBED_DIM), jnp.float32),
        mesh=mesh,
        compiler_params=pltpu.CompilerParams(
            use_tc_tiling_on_sc=False, needs_layout_passes=False),
        scratch_types=[
            pltpu.VMEM((CHUNK,), jnp.int32),
            pltpu.VMEM((CHUNK, EMBED_DIM), jnp.float32),
            pltpu.SemaphoreType.DMA,
            pltpu.SemaphoreType.DMA,
        ],
    )
    user_emb = sc(user_idx.astype(jnp.int32), user_table)
    return jnp.concatenate([user_emb, ts_emb, norm], axis=1)
